# baseline (speedup 1.0000x reference)
import dataclasses
import functools

import jax
import jax.numpy as jnp
from jax import lax
from jax.experimental import pallas as pl
from jax.experimental.pallas import tpu as pltpu
from jax.experimental.pallas import tpu_sc as plsc

_NUM_EXPERTS = 8
_TOPK = 2
_BLOCK = 4096
_NUM_CORES = 1
_NUM_SUBCORES = 16
_NW = _NUM_CORES * _NUM_SUBCORES
_LANES = 16


def _probs_block(x_ref, w_ref, b_ref, p_out):
    logits = jax.lax.dot_general(
        x_ref[...], w_ref[...], (((1,), (1,)), ((), ())),
        preferred_element_type=jnp.float32,
    ) + b_ref[...]
    p_out[...] = jax.nn.sigmoid(logits).T


def _tc_probs(hidden_states, w, b2):
    n_tokens, hidden = hidden_states.shape
    n_exp = w.shape[0]
    return pl.pallas_call(
        _probs_block,
        grid=(n_tokens // _BLOCK,),
        in_specs=[
            pl.BlockSpec((_BLOCK, hidden), lambda i: (i, 0)),
            pl.BlockSpec((n_exp, hidden), lambda i: (0, 0)),
            pl.BlockSpec((1, n_exp), lambda i: (0, 0)),
        ],
        out_specs=pl.BlockSpec((n_exp, _BLOCK), lambda i: (0, i)),
        out_shape=jax.ShapeDtypeStruct((n_exp, n_tokens), jnp.float32),
    )(hidden_states, w, b2)


def _sc_route(probs_t):
    n_exp, n_tokens = probs_t.shape
    tok_per_w = n_tokens // _NW
    n_groups = tok_per_w // _LANES
    mesh = plsc.VectorSubcoreMesh(core_axis_name="c", subcore_axis_name="s",
                                  num_cores=1)
    cp = pltpu.CompilerParams()
    if "needs_layout_passes" in pltpu.CompilerParams.__dataclass_fields__:
        cp = dataclasses.replace(cp, needs_layout_passes=False)

    half_tok = tok_per_w // 2
    half_groups = n_groups // 2
    half_elems = half_tok * _TOPK

    @functools.partial(
        pl.kernel,
        mesh=mesh,
        compiler_params=cp,
        out_type=[
            jax.ShapeDtypeStruct((n_tokens * _TOPK,), jnp.float32),
            jax.ShapeDtypeStruct((n_tokens * _TOPK,), jnp.int32),
        ],
        scratch_types=[
            pltpu.VMEM((_NUM_EXPERTS, tok_per_w), jnp.float32),
            pltpu.VMEM((tok_per_w * _TOPK,), jnp.float32),
            pltpu.VMEM((tok_per_w * _TOPK,), jnp.int32),
            pltpu.SemaphoreType.DMA,
            pltpu.SemaphoreType.DMA,
            pltpu.SemaphoreType.DMA,
        ],
    )
    def route(p_hbm, w_hbm, i_hbm, p_v, w_v, i_v, sem0, sem1, osem):
        wid = lax.axis_index("s") * _NUM_CORES + lax.axis_index("c")
        base = wid * tok_per_w

        cp0 = pltpu.make_async_copy(
            p_hbm.at[:, pl.ds(base, half_tok)],
            p_v.at[:, pl.ds(0, half_tok)], sem0)
        cp1 = pltpu.make_async_copy(
            p_hbm.at[:, pl.ds(base + half_tok, half_tok)],
            p_v.at[:, pl.ds(half_tok, half_tok)], sem1)
        cp0.start()
        cp1.start()

        def top2_group(g):
            t0 = g * _LANES
            m1 = p_v[0, pl.ds(t0, _LANES)]
            i1 = jnp.zeros((_LANES,), jnp.int32)
            m2 = jnp.full((_LANES,), -1.0, jnp.float32)
            i2 = jnp.zeros((_LANES,), jnp.int32)
            for e in range(1, _NUM_EXPERTS):
                v = p_v[e, pl.ds(t0, _LANES)]
                ecst = jnp.full((_LANES,), e, jnp.int32)
                gt1 = v > m1
                gt2 = v > m2
                m2n = jnp.where(gt1, m1, jnp.where(gt2, v, m2))
                i2n = jnp.where(gt1, i1, jnp.where(gt2, ecst, i2))
                m1 = jnp.where(gt1, v, m1)
                i1 = jnp.where(gt1, ecst, i1)
                m2 = m2n
                i2 = i2n
            s = m1 + m2
            off = (g >> 3) * (2 * 128) + (g & 7) * _LANES
            w_v[pl.ds(off, _LANES)] = m1 / s
            w_v[pl.ds(off + 128, _LANES)] = m2 / s
            i_v[pl.ds(off, _LANES)] = i1
            i_v[pl.ds(off + 128, _LANES)] = i2

        cp0.wait()

        @pl.loop(0, half_groups)
        def _(g):
            top2_group(g)

        ow0 = pltpu.make_async_copy(
            w_v.at[pl.ds(0, half_elems)],
            w_hbm.at[pl.ds(base * _TOPK, half_elems)], osem)
        oi0 = pltpu.make_async_copy(
            i_v.at[pl.ds(0, half_elems)],
            i_hbm.at[pl.ds(base * _TOPK, half_elems)], osem)
        ow0.start()
        oi0.start()
        cp1.wait()

        @pl.loop(half_groups, n_groups)
        def _(g):
            top2_group(g)

        ow1 = pltpu.make_async_copy(
            w_v.at[pl.ds(half_elems, half_elems)],
            w_hbm.at[pl.ds(base * _TOPK + half_elems, half_elems)], osem)
        oi1 = pltpu.make_async_copy(
            i_v.at[pl.ds(half_elems, half_elems)],
            i_hbm.at[pl.ds(base * _TOPK + half_elems, half_elems)], osem)
        ow1.start()
        oi1.start()
        ow0.wait()
        oi0.wait()
        ow1.wait()
        oi1.wait()

    return route(probs_t)


def kernel(hidden_states, W, b):
    n_tokens, hidden = hidden_states.shape
    n_exp = W.shape[0]
    b2 = b.reshape(1, n_exp)
    probs_t = _tc_probs(hidden_states, W, b2)
    w_flat, i_flat = _sc_route(probs_t)
    rw = (w_flat.reshape(n_tokens // 128, _TOPK, 128)
          .transpose(0, 2, 1).reshape(n_tokens, _TOPK))
    ri = (i_flat.reshape(n_tokens // 128, _TOPK, 128)
          .transpose(0, 2, 1).reshape(n_tokens, _TOPK))
    return (rw, ri, probs_t.T)


# speedup vs baseline: 1.0101x; 1.0101x over previous
import dataclasses
import functools

import jax
import jax.numpy as jnp
from jax import lax
from jax.experimental import pallas as pl
from jax.experimental.pallas import tpu as pltpu
from jax.experimental.pallas import tpu_sc as plsc

_NUM_EXPERTS = 8
_TOPK = 2
_BLOCK = 4096
_NUM_CORES = 2
_NUM_SUBCORES = 16
_NW = _NUM_CORES * _NUM_SUBCORES
_LANES = 16


def _probs_block(x_ref, w_ref, b_ref, p_out):
    logits = jax.lax.dot_general(
        x_ref[...], w_ref[...], (((1,), (1,)), ((), ())),
        preferred_element_type=jnp.float32,
    ) + b_ref[...]
    p_out[...] = jax.nn.sigmoid(logits).T


def _tc_probs(hidden_states, w, b2):
    n_tokens, hidden = hidden_states.shape
    n_exp = w.shape[0]
    return pl.pallas_call(
        _probs_block,
        grid=(n_tokens // _BLOCK,),
        in_specs=[
            pl.BlockSpec((_BLOCK, hidden), lambda i: (i, 0)),
            pl.BlockSpec((n_exp, hidden), lambda i: (0, 0)),
            pl.BlockSpec((1, n_exp), lambda i: (0, 0)),
        ],
        out_specs=pl.BlockSpec((n_exp, _BLOCK), lambda i: (0, i)),
        out_shape=jax.ShapeDtypeStruct((n_exp, n_tokens), jnp.float32),
    )(hidden_states, w, b2)


def _sc_route(probs_t):
    n_exp, n_tokens = probs_t.shape
    tok_per_w = n_tokens // _NW
    n_groups = tok_per_w // _LANES
    mesh = plsc.VectorSubcoreMesh(core_axis_name="c", subcore_axis_name="s")
    cp = pltpu.CompilerParams()
    if "needs_layout_passes" in pltpu.CompilerParams.__dataclass_fields__:
        cp = dataclasses.replace(cp, needs_layout_passes=False)

    half_tok = tok_per_w // 2
    half_groups = n_groups // 2
    half_elems = half_tok * _TOPK

    @functools.partial(
        pl.kernel,
        mesh=mesh,
        compiler_params=cp,
        out_type=[
            jax.ShapeDtypeStruct((n_tokens * _TOPK,), jnp.float32),
            jax.ShapeDtypeStruct((n_tokens * _TOPK,), jnp.int32),
        ],
        scratch_types=[
            pltpu.VMEM((_NUM_EXPERTS, tok_per_w), jnp.float32),
            pltpu.VMEM((tok_per_w * _TOPK,), jnp.float32),
            pltpu.VMEM((tok_per_w * _TOPK,), jnp.int32),
            pltpu.SemaphoreType.DMA,
            pltpu.SemaphoreType.DMA,
            pltpu.SemaphoreType.DMA,
        ],
    )
    def route(p_hbm, w_hbm, i_hbm, p_v, w_v, i_v, sem0, sem1, osem):
        wid = lax.axis_index("s") * _NUM_CORES + lax.axis_index("c")
        base = wid * tok_per_w

        cp0 = pltpu.make_async_copy(
            p_hbm.at[:, pl.ds(base, half_tok)],
            p_v.at[:, pl.ds(0, half_tok)], sem0)
        cp1 = pltpu.make_async_copy(
            p_hbm.at[:, pl.ds(base + half_tok, half_tok)],
            p_v.at[:, pl.ds(half_tok, half_tok)], sem1)
        cp0.start()
        cp1.start()

        def top2_group(g):
            t0 = g * _LANES
            m1 = p_v[0, pl.ds(t0, _LANES)]
            i1 = jnp.zeros((_LANES,), jnp.int32)
            m2 = jnp.full((_LANES,), -1.0, jnp.float32)
            i2 = jnp.zeros((_LANES,), jnp.int32)
            for e in range(1, _NUM_EXPERTS):
                v = p_v[e, pl.ds(t0, _LANES)]
                ecst = jnp.full((_LANES,), e, jnp.int32)
                gt1 = v > m1
                gt2 = v > m2
                m2n = jnp.where(gt1, m1, jnp.where(gt2, v, m2))
                i2n = jnp.where(gt1, i1, jnp.where(gt2, ecst, i2))
                m1 = jnp.where(gt1, v, m1)
                i1 = jnp.where(gt1, ecst, i1)
                m2 = m2n
                i2 = i2n
            s = m1 + m2
            off = (g >> 3) * (2 * 128) + (g & 7) * _LANES
            w_v[pl.ds(off, _LANES)] = m1 / s
            w_v[pl.ds(off + 128, _LANES)] = m2 / s
            i_v[pl.ds(off, _LANES)] = i1
            i_v[pl.ds(off + 128, _LANES)] = i2

        cp0.wait()

        @pl.loop(0, half_groups)
        def _(g):
            top2_group(g)

        ow0 = pltpu.make_async_copy(
            w_v.at[pl.ds(0, half_elems)],
            w_hbm.at[pl.ds(base * _TOPK, half_elems)], osem)
        oi0 = pltpu.make_async_copy(
            i_v.at[pl.ds(0, half_elems)],
            i_hbm.at[pl.ds(base * _TOPK, half_elems)], osem)
        ow0.start()
        oi0.start()
        cp1.wait()

        @pl.loop(half_groups, n_groups)
        def _(g):
            top2_group(g)

        ow1 = pltpu.make_async_copy(
            w_v.at[pl.ds(half_elems, half_elems)],
            w_hbm.at[pl.ds(base * _TOPK + half_elems, half_elems)], osem)
        oi1 = pltpu.make_async_copy(
            i_v.at[pl.ds(half_elems, half_elems)],
            i_hbm.at[pl.ds(base * _TOPK + half_elems, half_elems)], osem)
        ow1.start()
        oi1.start()
        ow0.wait()
        oi0.wait()
        ow1.wait()
        oi1.wait()

    return route(probs_t)


def kernel(hidden_states, W, b):
    n_tokens, hidden = hidden_states.shape
    n_exp = W.shape[0]
    b2 = b.reshape(1, n_exp)
    probs_t = _tc_probs(hidden_states, W, b2)
    w_flat, i_flat = _sc_route(probs_t)
    rw = (w_flat.reshape(n_tokens // 128, _TOPK, 128)
          .transpose(0, 2, 1).reshape(n_tokens, _TOPK))
    ri = (i_flat.reshape(n_tokens // 128, _TOPK, 128)
          .transpose(0, 2, 1).reshape(n_tokens, _TOPK))
    return (rw, ri, probs_t.T)
